# baseline (speedup 1.0000x reference)
import functools

import jax
import jax.numpy as jnp
from jax import lax
from jax.experimental import pallas as pl
from jax.experimental.pallas import tpu as pltpu
from jax.experimental.pallas import tpu_sc as plsc

_F32 = jnp.float32
NC = 2
NS = 16
LANES = 16
EB = 128
NP = 10240
BN = 512
ZB = 64
RT = NP // NS
DH = 128


def _ceil_div(a, b):
    return (a + b - 1) // b


def _fill_zeros(zb_v, width, rows):
    zeros = jnp.zeros((LANES,), _F32)

    def fill(r, carry):
        for k in range(width // LANES):
            zb_v[r, pl.ds(k * LANES, LANES)] = zeros
        return carry
    lax.fori_loop(0, rows, fill, 0)


def _zero_acc(zb_v, acc, s, rows):
    def zero(i, carry):
        pltpu.sync_copy(zb_v, acc.at[pl.ds(s * RT + i * rows, rows)])
        return carry
    lax.fori_loop(0, RT // rows, zero, 0)


@functools.lru_cache(maxsize=None)
def _make_deg(nb):
    mesh = plsc.VectorSubcoreMesh(
        core_axis_name="c", subcore_axis_name="s",
        num_cores=NC, num_subcores=NS)

    @functools.partial(
        pl.kernel,
        out_type=jax.ShapeDtypeStruct((NC * NP, DH), _F32),
        mesh=mesh,
        scratch_types=[
            pltpu.VMEM((nb, EB), jnp.int32),
            pltpu.VMEM((EB, DH), _F32),
            pltpu.VMEM((ZB, DH), _F32),
            pltpu.VMEM_SHARED((NP, DH), _F32),
        ],
    )
    def deg_kernel(dst_hbm, out_hbm, dst_v, ones_v, zb_v, acc):
        c = lax.axis_index("c")
        s = lax.axis_index("s")
        w = c * NS + s

        ones = jnp.ones((LANES,), _F32)

        def fill_ones(r, carry):
            for k in range(DH // LANES):
                ones_v[r, pl.ds(k * LANES, LANES)] = ones
            return carry
        lax.fori_loop(0, EB, fill_ones, 0)

        _fill_zeros(zb_v, DH, ZB)
        _zero_acc(zb_v, acc, s, ZB)

        pltpu.sync_copy(dst_hbm.at[pl.ds(w * nb, nb)], dst_v)
        plsc.subcore_barrier()

        def body(j, carry):
            pltpu.sync_copy(ones_v, acc.at[dst_v.at[j]], add=True)
            return carry
        lax.fori_loop(0, nb, body, 0)

        plsc.subcore_barrier()
        pltpu.sync_copy(acc.at[pl.ds(s * RT, RT)],
                        out_hbm.at[pl.ds(c * NP + s * RT, RT)])

    return deg_kernel


def _agg_loop(tbl, src_v, dst_v, rows0, rows1, acc, sem0, sem1, nb):
    def start(j, buf, sem):
        pltpu.async_copy(tbl.at[src_v.at[j]], buf, sem)

    def wait(buf, sem):
        pltpu.make_async_copy(tbl.at[src_v.at[0]], buf, sem).wait()

    start(0, rows0, sem0)

    def body(j2, carry):
        j = 2 * j2
        start(j + 1, rows1, sem1)
        wait(rows0, sem0)
        pltpu.sync_copy(rows0, acc.at[dst_v.at[j]], add=True)
        start(lax.rem(j + 2, nb), rows0, sem0)
        wait(rows1, sem1)
        pltpu.sync_copy(rows1, acc.at[dst_v.at[j + 1]], add=True)
        return carry
    lax.fori_loop(0, nb // 2, body, 0)
    wait(rows0, sem0)


_SC_MESH = dict(core_axis_name="c", subcore_axis_name="s",
                num_cores=NC, num_subcores=NS)


@functools.lru_cache(maxsize=None)
def _make_agg(nb):
    @functools.partial(
        pl.kernel,
        out_type=jax.ShapeDtypeStruct((NC * NP, DH), _F32),
        mesh=plsc.VectorSubcoreMesh(**_SC_MESH),
        scratch_types=[
            pltpu.VMEM((nb, EB), jnp.int32),
            pltpu.VMEM((nb, EB), jnp.int32),
            pltpu.VMEM((EB, DH), _F32),
            pltpu.VMEM((EB, DH), _F32),
            pltpu.VMEM_SHARED((NP, DH), _F32),
            pltpu.SemaphoreType.DMA,
            pltpu.SemaphoreType.DMA,
        ],
    )
    def agg_kernel(tbl_a, tbl_b, src_hbm, dst_hbm, out_hbm,
                   src_v, dst_v, rows0, rows1, acc, sem0, sem1):
        c = lax.axis_index("c")
        s = lax.axis_index("s")
        w = c * NS + s

        _fill_zeros(rows0, DH, EB)
        _zero_acc(rows0, acc, s, EB)
        pltpu.sync_copy(src_hbm.at[pl.ds(w * nb, nb)], src_v)
        pltpu.sync_copy(dst_hbm.at[pl.ds(w * nb, nb)], dst_v)
        plsc.subcore_barrier()

        @pl.when(c == 0)
        def _():
            _agg_loop(tbl_a, src_v, dst_v, rows0, rows1, acc,
                      sem0, sem1, nb)

        @pl.when(c == 1)
        def _():
            _agg_loop(tbl_b, src_v, dst_v, rows0, rows1, acc,
                      sem0, sem1, nb)

        plsc.subcore_barrier()
        pltpu.sync_copy(acc.at[pl.ds(s * RT, RT)],
                        out_hbm.at[pl.ds(c * NP + s * RT, RT)])

    return agg_kernel


def _dinv_block(dga_ref, dgb_ref):
    deg = dga_ref[:, 0:1] + dgb_ref[:, 0:1] + 1.0
    return lax.rsqrt(deg)


def _k1_body(x_ref, w_ref, dga_ref, dgb_ref, o_ref, o2_ref):
    dinv = _dinv_block(dga_ref, dgb_ref)
    g = dinv * jnp.dot(x_ref[...], w_ref[...], preferred_element_type=_F32)
    o_ref[...] = g
    o2_ref[...] = g


def _k_mid_body(pa_ref, pb_ref, g_ref, dga_ref, dgb_ref,
                b_ref, w_ref, o_ref, o2_ref):
    dinv = _dinv_block(dga_ref, dgb_ref)
    pre = dinv * (pa_ref[...] + pb_ref[...] + g_ref[...]) + b_ref[...]
    a = jnp.maximum(pre, 0.0)
    g = dinv * jnp.dot(a, w_ref[...], preferred_element_type=_F32)
    o_ref[...] = g
    o2_ref[...] = g


def _k_mid_split_body(pa_ref, pb_ref, g_ref, dga_ref, dgb_ref,
                      b_ref, w_ref, oa_ref, oa2_ref, ob_ref, ob2_ref):
    dinv = _dinv_block(dga_ref, dgb_ref)
    pre = dinv * (pa_ref[...] + pb_ref[...] + g_ref[...]) + b_ref[...]
    a = jnp.maximum(pre, 0.0)
    g = dinv * jnp.dot(a, w_ref[...], preferred_element_type=_F32)
    half = g.shape[1] // 2
    oa_ref[...] = g[:, :half]
    oa2_ref[...] = g[:, :half]
    ob_ref[...] = g[:, half:]
    ob2_ref[...] = g[:, half:]


def _k_final_body(aa0_ref, aa1_ref, ab0_ref, ab1_ref, ga_ref, gb_ref,
                  dga_ref, dgb_ref, b_ref, o_ref):
    dinv = _dinv_block(dga_ref, dgb_ref)
    left = aa0_ref[...] + aa1_ref[...] + ga_ref[...]
    right = ab0_ref[...] + ab1_ref[...] + gb_ref[...]
    o_ref[...] = dinv * jnp.concatenate([left, right], axis=1) + b_ref[...]


def _deg_specs():
    return [
        pl.BlockSpec((BN, DH), lambda i: (i, 0)),
        pl.BlockSpec((BN, DH), lambda i: (i + NP // BN, 0)),
    ]


def _part_specs(dh):
    return [
        pl.BlockSpec((BN, dh), lambda i: (i, 0)),
        pl.BlockSpec((BN, dh), lambda i: (i + NP // BN, 0)),
    ]


def kernel(x, edge_index, W1, b1, W2, b2, W3, b3):
    n, d_in = x.shape
    e = edge_index.shape[1]
    d_hid = W1.shape[1]
    d_out = W3.shape[1]
    grid = (_ceil_div(n, BN),)

    e_pad = _ceil_div(e, NC * NS * EB) * NC * NS * EB
    nb32 = e_pad // (NC * NS * EB)
    nb16 = e_pad // (NS * EB)
    src = jnp.concatenate(
        [edge_index[0], jnp.zeros((e_pad - e,), jnp.int32)]).reshape(-1, EB)
    dst = jnp.concatenate(
        [edge_index[1], jnp.full((e_pad - e,), n, jnp.int32)]).reshape(-1, EB)

    degp = _make_deg(nb32)(dst)

    table_specs = [
        pl.BlockSpec((BN, d_hid), lambda i: (i, 0)),
        pl.BlockSpec((BN, d_hid), lambda i: (i, 0)),
    ]
    table_shapes = [jax.ShapeDtypeStruct((n, d_hid), _F32)] * 2
    g1, g1d = pl.pallas_call(
        _k1_body,
        grid=grid,
        in_specs=[
            pl.BlockSpec((BN, d_in), lambda i: (i, 0)),
            pl.BlockSpec((d_in, d_hid), lambda i: (0, 0)),
            *_deg_specs(),
        ],
        out_specs=table_specs,
        out_shape=table_shapes,
    )(x, W1, degp, degp)

    agg1 = _make_agg(nb32)(g1, g1d, src, dst)

    g2, g2d = pl.pallas_call(
        _k_mid_body,
        grid=grid,
        in_specs=[
            *_part_specs(d_hid),
            pl.BlockSpec((BN, d_hid), lambda i: (i, 0)),
            *_deg_specs(),
            pl.BlockSpec((1, d_hid), lambda i: (0, 0)),
            pl.BlockSpec((d_hid, d_hid), lambda i: (0, 0)),
        ],
        out_specs=table_specs,
        out_shape=table_shapes,
    )(agg1, agg1, g1, degp, degp, b1.reshape(1, -1), W2)

    agg2 = _make_agg(nb32)(g2, g2d, src, dst)

    half3 = d_out // 2
    half_spec = pl.BlockSpec((BN, half3), lambda i: (i, 0))
    g3a, g3ad, g3b, g3bd = pl.pallas_call(
        _k_mid_split_body,
        grid=grid,
        in_specs=[
            *_part_specs(d_hid),
            pl.BlockSpec((BN, d_hid), lambda i: (i, 0)),
            *_deg_specs(),
            pl.BlockSpec((1, d_hid), lambda i: (0, 0)),
            pl.BlockSpec((d_hid, d_out), lambda i: (0, 0)),
        ],
        out_specs=[half_spec] * 4,
        out_shape=[jax.ShapeDtypeStruct((n, half3), _F32)] * 4,
    )(agg2, agg2, g2, degp, degp, b2.reshape(1, -1), W3)

    agg3a = _make_agg(nb32)(g3a, g3ad, src, dst)
    agg3b = _make_agg(nb32)(g3b, g3bd, src, dst)

    out = pl.pallas_call(
        _k_final_body,
        grid=grid,
        in_specs=[
            *_part_specs(half3),
            *_part_specs(half3),
            half_spec,
            half_spec,
            *_deg_specs(),
            pl.BlockSpec((1, d_out), lambda i: (0, 0)),
        ],
        out_specs=pl.BlockSpec((BN, d_out), lambda i: (i, 0)),
        out_shape=jax.ShapeDtypeStruct((n, d_out), _F32),
    )(agg3a, agg3a, agg3b, agg3b, g3a, g3b, degp, degp, b3.reshape(1, -1))

    return out


# speedup vs baseline: 3.0951x; 3.0951x over previous
import functools

import jax
import jax.numpy as jnp
from jax import lax
from jax.experimental import pallas as pl
from jax.experimental.pallas import tpu as pltpu
from jax.experimental.pallas import tpu_sc as plsc

_F32 = jnp.float32
NC = 2
NS = 16
LANES = 16
EB = 128
NP = 10240
BN = 512
ZB = 64
RT = NP // NS
DH = 128


def _ceil_div(a, b):
    return (a + b - 1) // b


def _fill_zeros(zb_v, width, rows):
    zeros = jnp.zeros((LANES,), _F32)

    def fill(r, carry):
        for k in range(width // LANES):
            zb_v[r, pl.ds(k * LANES, LANES)] = zeros
        return carry
    lax.fori_loop(0, rows, fill, 0)


def _zero_acc(zb_v, acc, s, rows):
    def zero(i, carry):
        pltpu.sync_copy(zb_v, acc.at[pl.ds(s * RT + i * rows, rows)])
        return carry
    lax.fori_loop(0, RT // rows, zero, 0)


@functools.lru_cache(maxsize=None)
def _make_deg(nb):
    mesh = plsc.VectorSubcoreMesh(
        core_axis_name="c", subcore_axis_name="s",
        num_cores=NC, num_subcores=NS)

    @functools.partial(
        pl.kernel,
        out_type=jax.ShapeDtypeStruct((NC * NP, DH), _F32),
        mesh=mesh,
        scratch_types=[
            pltpu.VMEM((nb, EB), jnp.int32),
            pltpu.VMEM((EB, DH), _F32),
            pltpu.VMEM((ZB, DH), _F32),
            pltpu.VMEM_SHARED((NP, DH), _F32),
        ],
    )
    def deg_kernel(dst_hbm, out_hbm, dst_v, ones_v, zb_v, acc):
        c = lax.axis_index("c")
        s = lax.axis_index("s")
        w = c * NS + s

        ones = jnp.ones((LANES,), _F32)

        def fill_ones(r, carry):
            for k in range(DH // LANES):
                ones_v[r, pl.ds(k * LANES, LANES)] = ones
            return carry
        lax.fori_loop(0, EB, fill_ones, 0)

        _fill_zeros(zb_v, DH, ZB)
        _zero_acc(zb_v, acc, s, ZB)

        pltpu.sync_copy(dst_hbm.at[pl.ds(w * nb, nb)], dst_v)
        plsc.subcore_barrier()

        def body(j, carry):
            pltpu.sync_copy(ones_v, acc.at[dst_v.at[j]], add=True)
            return carry
        lax.fori_loop(0, nb, body, 0)

        plsc.subcore_barrier()
        pltpu.sync_copy(acc.at[pl.ds(s * RT, RT)],
                        out_hbm.at[pl.ds(c * NP + s * RT, RT)])

    return deg_kernel


def _agg_loop(tbl, src_v, dst_v, rows0, rows1, acc, sem0, sem1, nb):
    def start(j, buf, sem):
        pltpu.async_copy(tbl.at[src_v.at[j]], buf, sem)

    def wait(buf, sem):
        pltpu.make_async_copy(tbl.at[src_v.at[0]], buf, sem).wait()

    start(0, rows0, sem0)

    def body(j2, carry):
        j = 2 * j2
        start(j + 1, rows1, sem1)
        wait(rows0, sem0)
        pltpu.sync_copy(rows0, acc.at[dst_v.at[j]], add=True)
        start(lax.rem(j + 2, nb), rows0, sem0)
        wait(rows1, sem1)
        pltpu.sync_copy(rows1, acc.at[dst_v.at[j + 1]], add=True)
        return carry
    lax.fori_loop(0, nb // 2, body, 0)
    wait(rows0, sem0)


_SC_MESH = dict(core_axis_name="c", subcore_axis_name="s",
                num_cores=NC, num_subcores=NS)


@functools.lru_cache(maxsize=None)
def _make_agg(nb):
    @functools.partial(
        pl.kernel,
        out_type=jax.ShapeDtypeStruct((NC * NP, DH), _F32),
        mesh=plsc.VectorSubcoreMesh(**_SC_MESH),
        scratch_types=[
            pltpu.VMEM((nb, EB), jnp.int32),
            pltpu.VMEM((nb, EB), jnp.int32),
            pltpu.VMEM((EB, DH), _F32),
            pltpu.VMEM((EB, DH), _F32),
            pltpu.VMEM_SHARED((NP, DH), _F32),
            pltpu.SemaphoreType.DMA,
            pltpu.SemaphoreType.DMA,
        ],
    )
    def agg_kernel(tbl_a, tbl_b, src_hbm, dst_hbm, out_hbm,
                   src_v, dst_v, rows0, rows1, acc, sem0, sem1):
        c = lax.axis_index("c")
        s = lax.axis_index("s")
        w = c * NS + s

        _fill_zeros(rows0, DH, EB)
        _zero_acc(rows0, acc, s, EB)
        pltpu.sync_copy(src_hbm.at[pl.ds(w * nb, nb)], src_v)
        pltpu.sync_copy(dst_hbm.at[pl.ds(w * nb, nb)], dst_v)
        plsc.subcore_barrier()

        @pl.when(c == 0)
        def _():
            _agg_loop(tbl_a, src_v, dst_v, rows0, rows1, acc,
                      sem0, sem1, nb)

        @pl.when(c == 1)
        def _():
            _agg_loop(tbl_b, src_v, dst_v, rows0, rows1, acc,
                      sem0, sem1, nb)

        plsc.subcore_barrier()
        pltpu.sync_copy(acc.at[pl.ds(s * RT, RT)],
                        out_hbm.at[pl.ds(c * NP + s * RT, RT)])

    return agg_kernel


def _dinv_block(dga_ref, dgb_ref):
    deg = dga_ref[:, 0:1] + dgb_ref[:, 0:1] + 1.0
    return lax.rsqrt(deg)


def _k1_body(x_ref, w_ref, dga_ref, dgb_ref, o_ref, o2_ref):
    dinv = _dinv_block(dga_ref, dgb_ref)
    g = dinv * jnp.dot(x_ref[...], w_ref[...], preferred_element_type=_F32)
    o_ref[...] = g
    o2_ref[...] = g


def _k_mid_body(pa_ref, pb_ref, g_ref, dga_ref, dgb_ref,
                b_ref, w_ref, o_ref, o2_ref):
    dinv = _dinv_block(dga_ref, dgb_ref)
    pre = dinv * (pa_ref[...] + pb_ref[...] + g_ref[...]) + b_ref[...]
    a = jnp.maximum(pre, 0.0)
    g = dinv * jnp.dot(a, w_ref[...], preferred_element_type=_F32)
    o_ref[...] = g
    o2_ref[...] = g


def _k_mid_split_body(pa_ref, pb_ref, g_ref, dga_ref, dgb_ref,
                      b_ref, w_ref, oa_ref, oa2_ref, ob_ref, ob2_ref):
    dinv = _dinv_block(dga_ref, dgb_ref)
    pre = dinv * (pa_ref[...] + pb_ref[...] + g_ref[...]) + b_ref[...]
    a = jnp.maximum(pre, 0.0)
    g = dinv * jnp.dot(a, w_ref[...], preferred_element_type=_F32)
    half = g.shape[1] // 2
    oa_ref[...] = g[:, :half]
    oa2_ref[...] = g[:, :half]
    ob_ref[...] = g[:, half:]
    ob2_ref[...] = g[:, half:]


def _k_final_body(aa0_ref, aa1_ref, ab0_ref, ab1_ref, ga_ref, gb_ref,
                  dga_ref, dgb_ref, b_ref, o_ref):
    dinv = _dinv_block(dga_ref, dgb_ref)
    left = aa0_ref[...] + aa1_ref[...] + ga_ref[...]
    right = ab0_ref[...] + ab1_ref[...] + gb_ref[...]
    o_ref[...] = dinv * jnp.concatenate([left, right], axis=1) + b_ref[...]


def _deg_specs():
    return [
        pl.BlockSpec((BN, DH), lambda i: (i, 0)),
        pl.BlockSpec((BN, DH), lambda i: (i + NP // BN, 0)),
    ]


def _part_specs(dh):
    return [
        pl.BlockSpec((BN, dh), lambda i: (i, 0)),
        pl.BlockSpec((BN, dh), lambda i: (i + NP // BN, 0)),
    ]


def kernel(x, edge_index, W1, b1, W2, b2, W3, b3):
    n, d_in = x.shape
    e = edge_index.shape[1]
    d_hid = W1.shape[1]
    d_out = W3.shape[1]
    grid = (_ceil_div(n, BN),)

    e_pad = _ceil_div(e, NC * NS * EB) * NC * NS * EB
    nb32 = e_pad // (NC * NS * EB)
    pad = jnp.arange(e_pad - e, dtype=jnp.int32)
    src = jnp.concatenate([edge_index[0], pad % n]).reshape(-1, EB)
    dst = jnp.concatenate(
        [edge_index[1], n + pad % (NP - n)]).reshape(-1, EB)

    degp = _make_deg(nb32)(dst)

    table_specs = [
        pl.BlockSpec((BN, d_hid), lambda i: (i, 0)),
        pl.BlockSpec((BN, d_hid), lambda i: (i, 0)),
    ]
    table_shapes = [jax.ShapeDtypeStruct((n, d_hid), _F32)] * 2
    g1, g1d = pl.pallas_call(
        _k1_body,
        grid=grid,
        in_specs=[
            pl.BlockSpec((BN, d_in), lambda i: (i, 0)),
            pl.BlockSpec((d_in, d_hid), lambda i: (0, 0)),
            *_deg_specs(),
        ],
        out_specs=table_specs,
        out_shape=table_shapes,
    )(x, W1, degp, degp)

    agg1 = _make_agg(nb32)(g1, g1d, src, dst)

    g2, g2d = pl.pallas_call(
        _k_mid_body,
        grid=grid,
        in_specs=[
            *_part_specs(d_hid),
            pl.BlockSpec((BN, d_hid), lambda i: (i, 0)),
            *_deg_specs(),
            pl.BlockSpec((1, d_hid), lambda i: (0, 0)),
            pl.BlockSpec((d_hid, d_hid), lambda i: (0, 0)),
        ],
        out_specs=table_specs,
        out_shape=table_shapes,
    )(agg1, agg1, g1, degp, degp, b1.reshape(1, -1), W2)

    agg2 = _make_agg(nb32)(g2, g2d, src, dst)

    half3 = d_out // 2
    half_spec = pl.BlockSpec((BN, half3), lambda i: (i, 0))
    g3a, g3ad, g3b, g3bd = pl.pallas_call(
        _k_mid_split_body,
        grid=grid,
        in_specs=[
            *_part_specs(d_hid),
            pl.BlockSpec((BN, d_hid), lambda i: (i, 0)),
            *_deg_specs(),
            pl.BlockSpec((1, d_hid), lambda i: (0, 0)),
            pl.BlockSpec((d_hid, d_out), lambda i: (0, 0)),
        ],
        out_specs=[half_spec] * 4,
        out_shape=[jax.ShapeDtypeStruct((n, half3), _F32)] * 4,
    )(agg2, agg2, g2, degp, degp, b2.reshape(1, -1), W3)

    agg3a = _make_agg(nb32)(g3a, g3ad, src, dst)
    agg3b = _make_agg(nb32)(g3b, g3bd, src, dst)

    out = pl.pallas_call(
        _k_final_body,
        grid=grid,
        in_specs=[
            *_part_specs(half3),
            *_part_specs(half3),
            half_spec,
            half_spec,
            *_deg_specs(),
            pl.BlockSpec((1, d_out), lambda i: (0, 0)),
        ],
        out_specs=pl.BlockSpec((BN, d_out), lambda i: (i, 0)),
        out_shape=jax.ShapeDtypeStruct((n, d_out), _F32),
    )(agg3a, agg3a, agg3b, agg3b, g3a, g3b, degp, degp, b3.reshape(1, -1))

    return out


# speedup vs baseline: 3.1393x; 1.0143x over previous
import functools

import jax
import jax.numpy as jnp
from jax import lax
from jax.experimental import pallas as pl
from jax.experimental.pallas import tpu as pltpu
from jax.experimental.pallas import tpu_sc as plsc

_F32 = jnp.float32
NC = 2
NS = 16
LANES = 16
EB = 128
NP = 10240
BN = 512
ZB = 64
RT = NP // NS
DH = 128


def _ceil_div(a, b):
    return (a + b - 1) // b


def _fill_zeros(zb_v, width, rows):
    zeros = jnp.zeros((LANES,), _F32)

    def fill(r, carry):
        for k in range(width // LANES):
            zb_v[r, pl.ds(k * LANES, LANES)] = zeros
        return carry
    lax.fori_loop(0, rows, fill, 0)


def _zero_acc(zb_v, acc, s, rows):
    def zero(i, carry):
        pltpu.sync_copy(zb_v, acc.at[pl.ds(s * RT + i * rows, rows)])
        return carry
    lax.fori_loop(0, RT // rows, zero, 0)


@functools.lru_cache(maxsize=None)
def _make_deg(nb):
    mesh = plsc.VectorSubcoreMesh(
        core_axis_name="c", subcore_axis_name="s",
        num_cores=NC, num_subcores=NS)

    @functools.partial(
        pl.kernel,
        out_type=jax.ShapeDtypeStruct((NC * NP, DH), _F32),
        mesh=mesh,
        scratch_types=[
            pltpu.VMEM((nb, EB), jnp.int32),
            pltpu.VMEM((EB, DH), _F32),
            pltpu.VMEM((ZB, DH), _F32),
            pltpu.VMEM_SHARED((NP, DH), _F32),
        ],
    )
    def deg_kernel(dst_hbm, out_hbm, dst_v, ones_v, zb_v, acc):
        c = lax.axis_index("c")
        s = lax.axis_index("s")
        w = c * NS + s

        ones = jnp.ones((LANES,), _F32)

        def fill_ones(r, carry):
            for k in range(DH // LANES):
                ones_v[r, pl.ds(k * LANES, LANES)] = ones
            return carry
        lax.fori_loop(0, EB, fill_ones, 0)

        _fill_zeros(zb_v, DH, ZB)
        _zero_acc(zb_v, acc, s, ZB)

        pltpu.sync_copy(dst_hbm.at[pl.ds(w * nb, nb)], dst_v)
        plsc.subcore_barrier()

        def body(j, carry):
            pltpu.sync_copy(ones_v, acc.at[dst_v.at[j]], add=True)
            return carry
        lax.fori_loop(0, nb, body, 0)

        plsc.subcore_barrier()
        pltpu.sync_copy(acc.at[pl.ds(s * RT, RT)],
                        out_hbm.at[pl.ds(c * NP + s * RT, RT)])

    return deg_kernel


def _agg_loop(tbl, src_v, dst_v, rows0, rows1, acc, sem0, sem1, nb):
    def start(j, buf, sem):
        pltpu.async_copy(tbl.at[src_v.at[j]], buf, sem)

    def wait(buf, sem):
        pltpu.make_async_copy(tbl.at[src_v.at[0]], buf, sem).wait()

    start(0, rows0, sem0)

    def body(j2, carry):
        j = 2 * j2
        start(j + 1, rows1, sem1)
        wait(rows0, sem0)
        pltpu.sync_copy(rows0, acc.at[dst_v.at[j]], add=True)
        start(lax.rem(j + 2, nb), rows0, sem0)
        wait(rows1, sem1)
        pltpu.sync_copy(rows1, acc.at[dst_v.at[j + 1]], add=True)
        return carry
    lax.fori_loop(0, nb // 2, body, 0)
    wait(rows0, sem0)


_SC_MESH = dict(core_axis_name="c", subcore_axis_name="s",
                num_cores=NC, num_subcores=NS)


@functools.lru_cache(maxsize=None)
def _make_agg(nb):
    @functools.partial(
        pl.kernel,
        out_type=jax.ShapeDtypeStruct((NC * NP, DH), _F32),
        mesh=plsc.VectorSubcoreMesh(**_SC_MESH),
        scratch_types=[
            pltpu.VMEM((nb, EB), jnp.int32),
            pltpu.VMEM((nb, EB), jnp.int32),
            pltpu.VMEM((EB, DH), _F32),
            pltpu.VMEM((EB, DH), _F32),
            pltpu.VMEM_SHARED((NP, DH), _F32),
            pltpu.SemaphoreType.DMA,
            pltpu.SemaphoreType.DMA,
        ],
    )
    def agg_kernel(tbl, src_hbm, dst_hbm, out_hbm,
                   src_v, dst_v, rows0, rows1, acc, sem0, sem1):
        c = lax.axis_index("c")
        s = lax.axis_index("s")
        w = c * NS + s

        _fill_zeros(rows0, DH, EB)
        _zero_acc(rows0, acc, s, EB)
        pltpu.sync_copy(src_hbm.at[pl.ds(w * nb, nb)], src_v)
        pltpu.sync_copy(dst_hbm.at[pl.ds(w * nb, nb)], dst_v)
        plsc.subcore_barrier()

        _agg_loop(tbl, src_v, dst_v, rows0, rows1, acc, sem0, sem1, nb)

        plsc.subcore_barrier()
        pltpu.sync_copy(acc.at[pl.ds(s * RT, RT)],
                        out_hbm.at[pl.ds(c * NP + s * RT, RT)])

    return agg_kernel


def _k1_body(x_ref, w_ref, dga_ref, dgb_ref, o_ref, dv_ref):
    deg = dga_ref[:, 0:1] + dgb_ref[:, 0:1] + 1.0
    dinv = lax.rsqrt(deg)
    o_ref[...] = dinv * jnp.dot(x_ref[...], w_ref[...],
                                preferred_element_type=_F32)
    dv_ref[...] = jnp.broadcast_to(dinv, (BN, DH))


def _dinv_col(dv_ref):
    return dv_ref[:, 0:1]


def _k_mid_body(pa_ref, pb_ref, g_ref, dv_ref, b_ref, w_ref, o_ref):
    dinv = _dinv_col(dv_ref)
    pre = dinv * (pa_ref[...] + pb_ref[...] + g_ref[...]) + b_ref[...]
    a = jnp.maximum(pre, 0.0)
    o_ref[...] = dinv * jnp.dot(a, w_ref[...], preferred_element_type=_F32)


def _k_mid_split_body(pa_ref, pb_ref, g_ref, dv_ref,
                      b_ref, w_ref, oa_ref, ob_ref):
    dinv = _dinv_col(dv_ref)
    pre = dinv * (pa_ref[...] + pb_ref[...] + g_ref[...]) + b_ref[...]
    a = jnp.maximum(pre, 0.0)
    g = dinv * jnp.dot(a, w_ref[...], preferred_element_type=_F32)
    half = g.shape[1] // 2
    oa_ref[...] = g[:, :half]
    ob_ref[...] = g[:, half:]


def _k_final_body(aa0_ref, aa1_ref, ab0_ref, ab1_ref, ga_ref, gb_ref,
                  dv_ref, b_ref, o_ref):
    dinv = _dinv_col(dv_ref)
    left = aa0_ref[...] + aa1_ref[...] + ga_ref[...]
    right = ab0_ref[...] + ab1_ref[...] + gb_ref[...]
    o_ref[...] = dinv * jnp.concatenate([left, right], axis=1) + b_ref[...]


def _deg_specs():
    return [
        pl.BlockSpec((BN, DH), lambda i: (i, 0)),
        pl.BlockSpec((BN, DH), lambda i: (i + NP // BN, 0)),
    ]


_DV_SPEC = pl.BlockSpec((BN, DH), lambda i: (i, 0))


def _part_specs(dh):
    return [
        pl.BlockSpec((BN, dh), lambda i: (i, 0)),
        pl.BlockSpec((BN, dh), lambda i: (i + NP // BN, 0)),
    ]


def kernel(x, edge_index, W1, b1, W2, b2, W3, b3):
    n, d_in = x.shape
    e = edge_index.shape[1]
    d_hid = W1.shape[1]
    d_out = W3.shape[1]
    grid = (_ceil_div(n, BN),)

    e_pad = _ceil_div(e, NC * NS * EB) * NC * NS * EB
    nb32 = e_pad // (NC * NS * EB)
    pad = jnp.arange(e_pad - e, dtype=jnp.int32)
    src = jnp.concatenate([edge_index[0], pad % n]).reshape(-1, EB)
    dst = jnp.concatenate(
        [edge_index[1], n + pad % (NP - n)]).reshape(-1, EB)

    degp = _make_deg(nb32)(dst)

    table_spec = pl.BlockSpec((BN, d_hid), lambda i: (i, 0))
    g1, dinv2d = pl.pallas_call(
        _k1_body,
        grid=grid,
        in_specs=[
            pl.BlockSpec((BN, d_in), lambda i: (i, 0)),
            pl.BlockSpec((d_in, d_hid), lambda i: (0, 0)),
            *_deg_specs(),
        ],
        out_specs=[table_spec, _DV_SPEC],
        out_shape=[
            jax.ShapeDtypeStruct((n, d_hid), _F32),
            jax.ShapeDtypeStruct((n, DH), _F32),
        ],
    )(x, W1, degp, degp)

    agg1 = _make_agg(nb32)(g1, src, dst)

    g2 = pl.pallas_call(
        _k_mid_body,
        grid=grid,
        in_specs=[
            *_part_specs(d_hid),
            table_spec,
            _DV_SPEC,
            pl.BlockSpec((1, d_hid), lambda i: (0, 0)),
            pl.BlockSpec((d_hid, d_hid), lambda i: (0, 0)),
        ],
        out_specs=table_spec,
        out_shape=jax.ShapeDtypeStruct((n, d_hid), _F32),
    )(agg1, agg1, g1, dinv2d, b1.reshape(1, -1), W2)

    agg2 = _make_agg(nb32)(g2, src, dst)

    half3 = d_out // 2
    half_spec = pl.BlockSpec((BN, half3), lambda i: (i, 0))
    g3a, g3b = pl.pallas_call(
        _k_mid_split_body,
        grid=grid,
        in_specs=[
            *_part_specs(d_hid),
            table_spec,
            _DV_SPEC,
            pl.BlockSpec((1, d_hid), lambda i: (0, 0)),
            pl.BlockSpec((d_hid, d_out), lambda i: (0, 0)),
        ],
        out_specs=[half_spec] * 2,
        out_shape=[jax.ShapeDtypeStruct((n, half3), _F32)] * 2,
    )(agg2, agg2, g2, dinv2d, b2.reshape(1, -1), W3)

    agg3a = _make_agg(nb32)(g3a, src, dst)
    agg3b = _make_agg(nb32)(g3b, src, dst)

    out = pl.pallas_call(
        _k_final_body,
        grid=grid,
        in_specs=[
            *_part_specs(half3),
            *_part_specs(half3),
            half_spec,
            half_spec,
            _DV_SPEC,
            pl.BlockSpec((1, d_out), lambda i: (0, 0)),
        ],
        out_specs=pl.BlockSpec((BN, d_out), lambda i: (i, 0)),
        out_shape=jax.ShapeDtypeStruct((n, d_out), _F32),
    )(agg3a, agg3a, agg3b, agg3b, g3a, g3b, dinv2d, b3.reshape(1, -1))

    return out
